# baseline (speedup 1.0000x reference)
import functools

import jax
import jax.numpy as jnp
from jax import lax
from jax.experimental import pallas as pl
from jax.experimental.pallas import tpu as pltpu
from jax.experimental.pallas import tpu_sc as plsc

D = 32
B = 16384
S = 50
NC, NS = 2, 16
NW = NC * NS
BW = B // NW
FW = BW * S
G = BW // 16


def _make_kernel():
    mesh = plsc.VectorSubcoreMesh(core_axis_name="c", subcore_axis_name="s")

    @functools.partial(
        pl.kernel,
        mesh=mesh,
        out_type=jax.ShapeDtypeStruct((S, D, B), jnp.float32),
        scratch_types=[
            pltpu.VMEM((FW,), jnp.int32),
            pltpu.VMEM((BW,), jnp.int32),
            pltpu.VMEM((BW,), jnp.int32),
            pltpu.VMEM((BW, D), jnp.float32),
            pltpu.VMEM((BW, D), jnp.float32),
            pltpu.VMEM((D, BW + 1), jnp.float32),
            pltpu.VMEM((D, BW + 1), jnp.float32),
            pltpu.SemaphoreType.DMA,
            pltpu.SemaphoreType.DMA,
            pltpu.SemaphoreType.DMA,
            pltpu.SemaphoreType.DMA,
        ],
        compiler_params=pltpu.CompilerParams(
            use_tc_tiling_on_sc=False, needs_layout_passes=False
        ),
    )
    def gather_kernel(idx_hbm, table_hbm, out_hbm, ids_v,
                      sidx0, sidx1, d0, d1, e0, e1, sem0, sem1, wsem0, wsem1):
        wid = lax.axis_index("s") * NC + lax.axis_index("c")
        b0 = wid * BW
        pltpu.sync_copy(idx_hbm.at[pl.ds(wid * FW, FW)], ids_v)

        iota = lax.iota(jnp.int32, 16)

        def out_slice(s):
            return out_hbm.at[s].at[:, pl.ds(b0, BW)]

        def compact(s, sidx):
            @plsc.parallel_loop(0, BW, step=16, unroll=2)
            def cbody(r0):
                rvec = iota + r0
                vis = plsc.load_gather(ids_v, [rvec * S + s])
                plsc.store_scatter(sidx, [rvec], vis)

        iota_hi = iota + 16

        def transpose(d_v, e_v):
            @plsc.parallel_loop(0, BW, step=8, unroll=4)
            def tbody(r0):
                for j in range(8):
                    r = r0 + j
                    rs = lax.broadcast(r, (16,))
                    x0 = d_v[r, pl.ds(0, 16)]
                    plsc.store_scatter(e_v, [iota, rs], x0)
                    x1 = d_v[r, pl.ds(16, 16)]
                    plsc.store_scatter(e_v, [iota_hi, rs], x1)

        compact(0, sidx0)
        pltpu.async_copy(table_hbm.at[sidx0], d0, sem0)

        def body(tt, carry):
            s = 2 * tt
            compact(s + 1, sidx1)
            pltpu.async_copy(table_hbm.at[sidx1], d1, sem1)
            pltpu.make_async_copy(table_hbm.at[sidx0], d0, sem0).wait()

            @pl.when(tt > 0)
            def _():
                pltpu.make_async_copy(e0.at[:, pl.ds(0, BW)], out_slice(0), wsem0).wait()

            transpose(d0, e0)
            pltpu.async_copy(e0.at[:, pl.ds(0, BW)], out_slice(s), wsem0)
            compact(s + 2, sidx0)
            pltpu.async_copy(table_hbm.at[sidx0], d0, sem0)
            pltpu.make_async_copy(table_hbm.at[sidx1], d1, sem1).wait()

            @pl.when(tt > 0)
            def _():
                pltpu.make_async_copy(e1.at[:, pl.ds(0, BW)], out_slice(0), wsem1).wait()

            transpose(d1, e1)
            pltpu.async_copy(e1.at[:, pl.ds(0, BW)], out_slice(s + 1), wsem1)
            return carry

        lax.fori_loop(0, S // 2 - 1, body, 0)
        compact(S - 1, sidx1)
        pltpu.async_copy(table_hbm.at[sidx1], d1, sem1)
        pltpu.make_async_copy(table_hbm.at[sidx0], d0, sem0).wait()
        pltpu.make_async_copy(e0.at[:, pl.ds(0, BW)], out_slice(0), wsem0).wait()
        transpose(d0, e0)
        pltpu.async_copy(e0.at[:, pl.ds(0, BW)], out_slice(S - 2), wsem0)
        pltpu.make_async_copy(table_hbm.at[sidx1], d1, sem1).wait()
        pltpu.make_async_copy(e1.at[:, pl.ds(0, BW)], out_slice(0), wsem1).wait()
        transpose(d1, e1)
        pltpu.async_copy(e1.at[:, pl.ds(0, BW)], out_slice(S - 1), wsem1)
        pltpu.make_async_copy(e0.at[:, pl.ds(0, BW)], out_slice(0), wsem0).wait()
        pltpu.make_async_copy(e1.at[:, pl.ds(0, BW)], out_slice(0), wsem1).wait()

    return gather_kernel


_GATHER = _make_kernel()


def kernel(element_ids, weight):
    idx = element_ids.reshape(-1)
    out = _GATHER(idx, weight)
    return out.transpose(2, 0, 1)


# speedup vs baseline: 1.0273x; 1.0273x over previous
import functools

import jax
import jax.numpy as jnp
from jax import lax
from jax.experimental import pallas as pl
from jax.experimental.pallas import tpu as pltpu
from jax.experimental.pallas import tpu_sc as plsc

D = 32
B = 16384
S = 50
NC, NS = 2, 16
NW = NC * NS
BW = B // NW
FW = BW * S
G = BW // 16


def _make_kernel():
    mesh = plsc.VectorSubcoreMesh(core_axis_name="c", subcore_axis_name="s")

    @functools.partial(
        pl.kernel,
        mesh=mesh,
        out_type=jax.ShapeDtypeStruct((S, D, B), jnp.float32),
        scratch_types=[
            pltpu.VMEM((FW,), jnp.int32),
            pltpu.VMEM((BW,), jnp.int32),
            pltpu.VMEM((BW,), jnp.int32),
            pltpu.VMEM((BW, D), jnp.float32),
            pltpu.VMEM((BW, D), jnp.float32),
            pltpu.VMEM((D, BW + 1), jnp.float32),
            pltpu.VMEM((D, BW + 1), jnp.float32),
            pltpu.SemaphoreType.DMA,
            pltpu.SemaphoreType.DMA,
            pltpu.SemaphoreType.DMA,
            pltpu.SemaphoreType.DMA,
        ],
        compiler_params=pltpu.CompilerParams(
            use_tc_tiling_on_sc=False, needs_layout_passes=False
        ),
    )
    def gather_kernel(idx_hbm, table_hbm, out_hbm, ids_v,
                      sidx0, sidx1, d0, d1, e0, e1, sem0, sem1, wsem0, wsem1):
        wid = lax.axis_index("s") * NC + lax.axis_index("c")
        b0 = wid * BW
        pltpu.sync_copy(idx_hbm.at[pl.ds(wid * FW, FW)], ids_v)

        iota = lax.iota(jnp.int32, 16)

        def out_slice(s):
            return out_hbm.at[s].at[:, pl.ds(b0, BW)]

        def compact(s, sidx):
            @plsc.parallel_loop(0, BW, step=16, unroll=2)
            def cbody(r0):
                rvec = iota + r0
                vis = plsc.load_gather(ids_v, [rvec * S + s])
                plsc.store_scatter(sidx, [rvec], vis)

        iota_hi = iota + 16

        def transpose(d_v, e_v):
            @plsc.parallel_loop(0, BW, step=8, unroll=2)
            def tbody(r0):
                for j in range(8):
                    r = r0 + j
                    rs = lax.broadcast(r, (16,))
                    x0 = d_v[r, pl.ds(0, 16)]
                    plsc.store_scatter(e_v, [iota, rs], x0)
                    x1 = d_v[r, pl.ds(16, 16)]
                    plsc.store_scatter(e_v, [iota_hi, rs], x1)

        compact(0, sidx0)
        pltpu.async_copy(table_hbm.at[sidx0], d0, sem0)

        def body(tt, carry):
            s = 2 * tt
            compact(s + 1, sidx1)
            pltpu.async_copy(table_hbm.at[sidx1], d1, sem1)
            pltpu.make_async_copy(table_hbm.at[sidx0], d0, sem0).wait()

            @pl.when(tt > 0)
            def _():
                pltpu.make_async_copy(e0.at[:, pl.ds(0, BW)], out_slice(0), wsem0).wait()

            transpose(d0, e0)
            pltpu.async_copy(e0.at[:, pl.ds(0, BW)], out_slice(s), wsem0)
            compact(s + 2, sidx0)
            pltpu.async_copy(table_hbm.at[sidx0], d0, sem0)
            pltpu.make_async_copy(table_hbm.at[sidx1], d1, sem1).wait()

            @pl.when(tt > 0)
            def _():
                pltpu.make_async_copy(e1.at[:, pl.ds(0, BW)], out_slice(0), wsem1).wait()

            transpose(d1, e1)
            pltpu.async_copy(e1.at[:, pl.ds(0, BW)], out_slice(s + 1), wsem1)
            return carry

        lax.fori_loop(0, S // 2 - 1, body, 0)
        compact(S - 1, sidx1)
        pltpu.async_copy(table_hbm.at[sidx1], d1, sem1)
        pltpu.make_async_copy(table_hbm.at[sidx0], d0, sem0).wait()
        pltpu.make_async_copy(e0.at[:, pl.ds(0, BW)], out_slice(0), wsem0).wait()
        transpose(d0, e0)
        pltpu.async_copy(e0.at[:, pl.ds(0, BW)], out_slice(S - 2), wsem0)
        pltpu.make_async_copy(table_hbm.at[sidx1], d1, sem1).wait()
        pltpu.make_async_copy(e1.at[:, pl.ds(0, BW)], out_slice(0), wsem1).wait()
        transpose(d1, e1)
        pltpu.async_copy(e1.at[:, pl.ds(0, BW)], out_slice(S - 1), wsem1)
        pltpu.make_async_copy(e0.at[:, pl.ds(0, BW)], out_slice(0), wsem0).wait()
        pltpu.make_async_copy(e1.at[:, pl.ds(0, BW)], out_slice(0), wsem1).wait()

    return gather_kernel


_GATHER = _make_kernel()


def kernel(element_ids, weight):
    idx = element_ids.reshape(-1)
    out = _GATHER(idx, weight)
    return out.transpose(2, 0, 1)


# speedup vs baseline: 1.0288x; 1.0015x over previous
import functools

import jax
import jax.numpy as jnp
from jax import lax
from jax.experimental import pallas as pl
from jax.experimental.pallas import tpu as pltpu
from jax.experimental.pallas import tpu_sc as plsc

D = 32
B = 16384
S = 50
NC, NS = 2, 16
NW = NC * NS
BW = B // NW
FW = BW * S


def _make_kernel():
    mesh = plsc.VectorSubcoreMesh(core_axis_name="c", subcore_axis_name="s")

    @functools.partial(
        pl.kernel,
        mesh=mesh,
        out_type=jax.ShapeDtypeStruct((S, D, B), jnp.float32),
        scratch_types=[
            pltpu.VMEM((FW,), jnp.int32),
            pltpu.VMEM((BW,), jnp.int32),
            pltpu.VMEM((BW,), jnp.int32),
            pltpu.VMEM((BW, D), jnp.float32),
            pltpu.VMEM((BW, D), jnp.float32),
            pltpu.VMEM((D, BW + 1), jnp.float32),
            pltpu.VMEM((D, BW + 1), jnp.float32),
            pltpu.SemaphoreType.DMA,
            pltpu.SemaphoreType.DMA,
            pltpu.SemaphoreType.DMA,
            pltpu.SemaphoreType.DMA,
        ],
        compiler_params=pltpu.CompilerParams(
            use_tc_tiling_on_sc=False, needs_layout_passes=False
        ),
    )
    def gather_kernel(idx_hbm, table_hbm, out_hbm, ids_v,
                      sidx0, sidx1, d0, d1, e0, e1, sem0, sem1, wsem0, wsem1):
        wid = lax.axis_index("s") * NC + lax.axis_index("c")
        b0 = wid * BW
        pltpu.sync_copy(idx_hbm.at[pl.ds(wid * FW, FW)], ids_v)

        iota = lax.iota(jnp.int32, 16)

        def out_slice(s):
            return out_hbm.at[s].at[:, pl.ds(b0, BW)]

        def compact(s, sidx):
            @plsc.parallel_loop(0, BW, step=16, unroll=2)
            def cbody(r0):
                rvec = iota + r0
                vis = plsc.load_gather(ids_v, [rvec * S + s])
                plsc.store_scatter(sidx, [rvec], vis)

        iota_hi = iota + 16

        def transpose(d_v, e_v):
            @plsc.parallel_loop(0, BW, step=8, unroll=2)
            def tbody(r0):
                for j in range(8):
                    r = r0 + j
                    rs = lax.broadcast(r, (16,))
                    x0 = d_v[r, pl.ds(0, 16)]
                    plsc.store_scatter(e_v, [iota, rs], x0)
                    x1 = d_v[r, pl.ds(16, 16)]
                    plsc.store_scatter(e_v, [iota_hi, rs], x1)

        compact(0, sidx0)
        pltpu.async_copy(table_hbm.at[sidx0], d0, sem0)

        def body(tt, carry):
            s = 2 * tt
            compact(s + 1, sidx1)
            pltpu.async_copy(table_hbm.at[sidx1], d1, sem1)
            pltpu.make_async_copy(table_hbm.at[sidx0], d0, sem0).wait()

            @pl.when(tt > 0)
            def _():
                pltpu.make_async_copy(e0.at[:, pl.ds(0, BW)], out_slice(0), wsem0).wait()

            transpose(d0, e0)
            pltpu.async_copy(e0.at[:, pl.ds(0, BW)], out_slice(s), wsem0)
            compact(s + 2, sidx0)
            pltpu.async_copy(table_hbm.at[sidx0], d0, sem0)
            pltpu.make_async_copy(table_hbm.at[sidx1], d1, sem1).wait()

            @pl.when(tt > 0)
            def _():
                pltpu.make_async_copy(e1.at[:, pl.ds(0, BW)], out_slice(0), wsem1).wait()

            transpose(d1, e1)
            pltpu.async_copy(e1.at[:, pl.ds(0, BW)], out_slice(s + 1), wsem1)
            return carry

        lax.fori_loop(0, S // 2 - 1, body, 0)
        compact(S - 1, sidx1)
        pltpu.async_copy(table_hbm.at[sidx1], d1, sem1)
        pltpu.make_async_copy(table_hbm.at[sidx0], d0, sem0).wait()
        pltpu.make_async_copy(e0.at[:, pl.ds(0, BW)], out_slice(0), wsem0).wait()
        transpose(d0, e0)
        pltpu.async_copy(e0.at[:, pl.ds(0, BW)], out_slice(S - 2), wsem0)
        pltpu.make_async_copy(table_hbm.at[sidx1], d1, sem1).wait()
        pltpu.make_async_copy(e1.at[:, pl.ds(0, BW)], out_slice(0), wsem1).wait()
        transpose(d1, e1)
        pltpu.async_copy(e1.at[:, pl.ds(0, BW)], out_slice(S - 1), wsem1)
        pltpu.make_async_copy(e0.at[:, pl.ds(0, BW)], out_slice(0), wsem0).wait()
        pltpu.make_async_copy(e1.at[:, pl.ds(0, BW)], out_slice(0), wsem1).wait()

    return gather_kernel


_GATHER = _make_kernel()


def kernel(element_ids, weight):
    idx = element_ids.reshape(-1)
    out = _GATHER(idx, weight)
    return out.transpose(2, 0, 1)
